# Initial kernel scaffold; baseline (speedup 1.0000x reference)
import jax
import jax.numpy as jnp
from jax.experimental import pallas as pl


def kernel(x, edge_index, W1, b1, W2, b2, W3, b3):
    raise NotImplementedError("write your pallas kernel here")


# speedup vs baseline: 38.3447x; 38.3447x over previous
import functools

import jax
import jax.numpy as jnp
from jax import lax
from jax.experimental import pallas as pl
from jax.experimental.pallas import tpu as pltpu
from jax.experimental.pallas import tpu_sc as plsc

N = 10000
NP = 10240
E = 320000
D = 128
F = 64

NC = 2
NS = 16
NW = NC * NS
K = 80
EPT = E // NW
NBLK = EPT // K
NBUF = 5
NGRP = NBLK // NBUF
ZPT = NP // NS

_mesh = plsc.VectorSubcoreMesh(core_axis_name="c", subcore_axis_name="s")


def _deg_body(col_hbm, ones_hbm, zeros_hbm, degp_hbm, deg_sh, idx_v, ones_v, sem):
    c = lax.axis_index("c")
    s = lax.axis_index("s")
    wid = c * NS + s
    pltpu.sync_copy(zeros_hbm.at[pl.ds(s * ZPT, ZPT)], deg_sh.at[pl.ds(s * ZPT, ZPT)])
    pltpu.sync_copy(ones_hbm, ones_v)
    pltpu.sync_copy(col_hbm.at[wid], idx_v)
    plsc.subcore_barrier()

    def body(j, carry):
        pltpu.sync_copy(ones_v, deg_sh.at[idx_v.at[j]], add=True)
        return carry

    lax.fori_loop(0, NBLK, body, 0)
    plsc.subcore_barrier()
    pltpu.sync_copy(deg_sh.at[pl.ds(s * ZPT, ZPT)],
                    degp_hbm.at[c].at[pl.ds(s * ZPT, ZPT)])


_deg_call = functools.partial(
    pl.kernel,
    out_type=jax.ShapeDtypeStruct((NC, NP), jnp.float32),
    mesh=_mesh,
    scratch_types=[
        pltpu.VMEM_SHARED((NP,), jnp.float32),
        pltpu.VMEM((NBLK, K), jnp.int32),
        pltpu.VMEM((K,), jnp.float32),
        pltpu.SemaphoreType.DMA,
    ],
)(_deg_body)


def _scat_body(hp_hbm, row_hbm, col_hbm, zeros_hbm, out_hbm,
               acc_sh, ridx, cidx, gbuf, *sems):
    gsems, ssems = sems[:NBUF], sems[NBUF:]
    c = lax.axis_index("c")
    s = lax.axis_index("s")
    wid = c * NS + s

    @pl.when(c == 0)
    def _():
        pltpu.sync_copy(hp_hbm.at[pl.ds(s * ZPT, ZPT)],
                        acc_sh.at[pl.ds(s * ZPT, ZPT)])

    @pl.when(c != 0)
    def _():
        pltpu.sync_copy(zeros_hbm.at[pl.ds(s * ZPT, ZPT)],
                        acc_sh.at[pl.ds(s * ZPT, ZPT)])

    pltpu.sync_copy(row_hbm.at[wid], ridx)
    pltpu.sync_copy(col_hbm.at[wid], cidx)
    plsc.subcore_barrier()

    for b in range(NBUF):
        pltpu.async_copy(hp_hbm.at[ridx.at[b]], gbuf.at[b], gsems[b])

    def group(g, carry):
        for b in range(NBUF):
            j = g * NBUF + b
            pltpu.make_async_copy(hp_hbm.at[ridx.at[j]], gbuf.at[b], gsems[b]).wait()
            pltpu.async_copy(gbuf.at[b], acc_sh.at[cidx.at[j]], ssems[b], add=True)

            @pl.when(g < NGRP - 1)
            def _():
                pltpu.make_async_copy(gbuf.at[b], acc_sh.at[cidx.at[j]],
                                      ssems[b]).wait()
                pltpu.async_copy(hp_hbm.at[ridx.at[j + NBUF]], gbuf.at[b], gsems[b])

        return carry

    lax.fori_loop(0, NGRP, group, 0)
    for b in range(NBUF):
        j = (NGRP - 1) * NBUF + b
        pltpu.make_async_copy(gbuf.at[b], acc_sh.at[cidx.at[j]], ssems[b]).wait()
    plsc.subcore_barrier()
    pltpu.sync_copy(acc_sh.at[pl.ds(s * ZPT, ZPT)],
                    out_hbm.at[c].at[pl.ds(s * ZPT, ZPT)])


_scat_call = functools.partial(
    pl.kernel,
    out_type=jax.ShapeDtypeStruct((NC, NP, F), jnp.float32),
    mesh=_mesh,
    scratch_types=[
        pltpu.VMEM_SHARED((NP, F), jnp.float32),
        pltpu.VMEM((NBLK, K), jnp.int32),
        pltpu.VMEM((NBLK, K), jnp.int32),
        pltpu.VMEM((NBUF, K, F), jnp.float32),
    ] + [pltpu.SemaphoreType.DMA] * (2 * NBUF),
    compiler_params=pltpu.CompilerParams(use_tc_tiling_on_sc=False),
)(_scat_body)


def _dis_body(dp_ref, o_ref):
    o_ref[...] = lax.rsqrt(dp_ref[0] + dp_ref[1] + 1.0)


def _dis_call(degp):
    return pl.pallas_call(
        _dis_body,
        out_shape=jax.ShapeDtypeStruct((NP,), jnp.float32),
    )(degp)


_RB = 640
_NRB = NP // _RB


def _mm1_body(x_ref, w_ref, dis_ref, o_ref):
    h = lax.dot_general(x_ref[...], w_ref[...], (((1,), (1,)), ((), ())),
                        preferred_element_type=jnp.float32)
    o_ref[...] = h * dis_ref[...]


def _mm1_call(x_p, W1, dis2):
    return pl.pallas_call(
        _mm1_body,
        grid=(_NRB,),
        in_specs=[
            pl.BlockSpec((_RB, D), lambda i: (i, 0)),
            pl.BlockSpec((F, D), lambda i: (0, 0)),
            pl.BlockSpec((_RB, 1), lambda i: (i, 0)),
        ],
        out_specs=pl.BlockSpec((_RB, F), lambda i: (i, 0)),
        out_shape=jax.ShapeDtypeStruct((NP, F), jnp.float32),
    )(x_p, W1, dis2)


def _epimm_body(p_ref, dis_ref, b_ref, w_ref, o_ref):
    g = jax.nn.sigmoid(dis_ref[...] * (p_ref[0] + p_ref[1]) + b_ref[...])
    h = lax.dot_general(g, w_ref[...], (((1,), (1,)), ((), ())),
                        preferred_element_type=jnp.float32)
    o_ref[...] = h * dis_ref[...]


def _epimm_call(parts, dis2, b, Wn):
    return pl.pallas_call(
        _epimm_body,
        grid=(_NRB,),
        in_specs=[
            pl.BlockSpec((NC, _RB, F), lambda i: (0, i, 0)),
            pl.BlockSpec((_RB, 1), lambda i: (i, 0)),
            pl.BlockSpec((1, F), lambda i: (0, 0)),
            pl.BlockSpec((F, F), lambda i: (0, 0)),
        ],
        out_specs=pl.BlockSpec((_RB, F), lambda i: (i, 0)),
        out_shape=jax.ShapeDtypeStruct((NP, F), jnp.float32),
    )(parts, dis2, b, Wn)


_MB = 400
_NMB = N // _MB


def _mean_body(p_ref, dis_ref, b3_ref, o_ref):
    i = pl.program_id(0)

    @pl.when(i == 0)
    def _():
        o_ref[...] = jnp.zeros_like(o_ref)

    o_ref[...] += jnp.sum(dis_ref[...] * (p_ref[0] + p_ref[1]), axis=0,
                          keepdims=True)

    @pl.when(i == _NMB - 1)
    def _():
        o_ref[...] = o_ref[...] * (1.0 / N) + b3_ref[...]


def _mean_call(parts, dis2, b3):
    return pl.pallas_call(
        _mean_body,
        grid=(_NMB,),
        in_specs=[
            pl.BlockSpec((NC, _MB, F), lambda i: (0, i, 0)),
            pl.BlockSpec((_MB, 1), lambda i: (i, 0)),
            pl.BlockSpec((1, F), lambda i: (0, 0)),
        ],
        out_specs=pl.BlockSpec((1, F), lambda i: (0, 0)),
        out_shape=jax.ShapeDtypeStruct((1, F), jnp.float32),
    )(parts, dis2, b3)


def kernel(x, edge_index, W1, b1, W2, b2, W3, b3):
    row = edge_index[0].reshape(NW, NBLK, K)
    col = edge_index[1].reshape(NW, NBLK, K)
    x_p = jnp.zeros((NP, D), jnp.float32).at[:N].set(x)
    zeros2 = jnp.zeros((NP, F), jnp.float32)
    zeros1 = jnp.zeros((NP,), jnp.float32)
    ones_k = jnp.ones((K,), jnp.float32)

    degp = _deg_call(col, ones_k, zeros1)
    dis2 = _dis_call(degp).reshape(NP, 1)

    hp = _mm1_call(x_p, W1, dis2)
    parts = _scat_call(hp, row, col, zeros2)
    hp = _epimm_call(parts, dis2, b1.reshape(1, F), W2)
    parts = _scat_call(hp, row, col, zeros2)
    hp = _epimm_call(parts, dis2, b2.reshape(1, F), W3)
    parts = _scat_call(hp, row, col, zeros2)
    out = _mean_call(parts, dis2, b3.reshape(1, F))
    return out.reshape(F)
